# baseline (speedup 1.0000x reference)
import functools

import jax
import jax.numpy as jnp
from jax import lax
from jax.experimental import pallas as pl
from jax.experimental.pallas import tpu as pltpu
from jax.experimental.pallas import tpu_sc as plsc

B = 4096
DU = 768
DA = 256
HEADS = 8
NW = 32
ROWS_W = B // NW
CH = 32
NCH = ROWS_W // CH


def _sc_body(xt_hbm, rew_hbm, subt_hbm, act_hbm, out_hbm,
             xtbuf, actrep, zrow, rew_v, sub_v, in_sem, out_sem):
    wid = lax.axis_index("c") * 16 + lax.axis_index("s")
    base = wid * ROWS_W

    pltpu.sync_copy(rew_hbm.at[pl.ds(base, ROWS_W)], rew_v)
    pltpu.sync_copy(subt_hbm.at[:, pl.ds(base, ROWS_W)], sub_v)

    def zr_body(v, _):
        zrow[pl.ds(v * 16, 16)] = jnp.zeros((16,), jnp.float32)
        return 0
    lax.fori_loop(0, DU // 16, zr_body, 0)

    def rep_body(i, _):
        h = lax.div(i, CH)
        r = lax.rem(i, CH)
        pltpu.sync_copy(act_hbm.at[h], actrep.at[h, r])
        return 0
    lax.fori_loop(0, HEADS * CH, rep_body, 0)

    def stage(c, slot):
        row0 = base + c * CH
        return pltpu.async_copy(
            xt_hbm.at[pl.ds(row0, CH)], xtbuf.at[slot], in_sem)

    def fire(c, slot):
        row0 = base + c * CH
        handles = []
        for h in range(HEADS):
            handles.append(pltpu.async_copy(
                xtbuf.at[slot],
                out_hbm.at[h, pl.ds(row0, CH), pl.ds(0, DU)], out_sem))
            handles.append(pltpu.async_copy(
                actrep.at[h],
                out_hbm.at[h, pl.ds(row0, CH), pl.ds(DU, DA)], out_sem))
        return handles

    pending = [None, None]
    stage(0, 0).wait()
    pending[0] = fire(0, 0)
    for c in range(1, NCH):
        slot = c % 2
        if pending[slot] is not None:
            for hnd in pending[slot]:
                hnd.wait()
            pending[slot] = None
        stage(c, slot).wait()
        pending[slot] = fire(c, slot)
    for p in pending:
        if p is not None:
            for hnd in p:
                hnd.wait()

    def sweep(i, _):
        c = lax.div(i, HEADS)
        h = lax.rem(i, HEADS)
        off = c * CH
        row0 = base + off
        rew_a = rew_v[pl.ds(off, 16)]
        rew_b = rew_v[pl.ds(off + 16, 16)]
        sub_a = sub_v[h, pl.ds(off, 16)]
        sub_b = sub_v[h, pl.ds(off + 16, 16)]
        mfa = jnp.where((rew_a == 1) & (sub_a >= 0.1), 1.0, 0.0)
        mfb = jnp.where((rew_b == 1) & (sub_b >= 0.1), 1.0, 0.0)
        nsel = jnp.sum(mfa) + jnp.sum(mfb)

        @pl.when(nsel < CH - 0.5)
        def _fix():
            def rowfn(r, _):
                lane = lax.rem(r, 16)
                sel_a = jnp.max(
                    jnp.where(lax.iota(jnp.int32, 16) == lane, mfa, 0.0))
                sel_b = jnp.max(
                    jnp.where(lax.iota(jnp.int32, 16) == lane, mfb, 0.0))
                mr = jnp.where(r < 16, sel_a, sel_b)

                @pl.when(mr < 0.5)
                def _zero():
                    pltpu.sync_copy(
                        zrow, out_hbm.at[h, row0 + r, pl.ds(0, DU)])
                return 0
            lax.fori_loop(0, CH, rowfn, 0)
        return 0

    lax.fori_loop(0, NCH * HEADS, sweep, 0)


_sc_call = functools.partial(
    pl.kernel,
    out_type=jax.ShapeDtypeStruct((HEADS, B, DU + DA), jnp.float32),
    mesh=plsc.VectorSubcoreMesh(core_axis_name="c", subcore_axis_name="s"),
    compiler_params=pltpu.CompilerParams(needs_layout_passes=False),
    scratch_types=[
        pltpu.VMEM((2, CH, DU), jnp.float32),
        pltpu.VMEM((HEADS, CH, DA), jnp.float32),
        pltpu.VMEM((DU,), jnp.float32),
        pltpu.VMEM((ROWS_W,), jnp.int32),
        pltpu.VMEM((HEADS, ROWS_W), jnp.float32),
        pltpu.SemaphoreType.DMA,
        pltpu.SemaphoreType.DMA,
    ],
)(_sc_body)


def kernel(xt, rewards, subset, action):
    xt2 = xt.reshape(B, DU)
    subt = subset.T
    out = _sc_call(xt2, rewards, subt, action)
    return out.reshape(HEADS, B, 1, DU + DA)


# speedup vs baseline: 2.7063x; 2.7063x over previous
import functools

import jax
import jax.numpy as jnp
from jax import lax
from jax.experimental import pallas as pl
from jax.experimental.pallas import tpu as pltpu
from jax.experimental.pallas import tpu_sc as plsc

B = 4096
DU = 768
DA = 256
HEADS = 8
NW = 32
ROWS_W = B // NW
CH = 32
NCH = ROWS_W // CH


def _sc_body(xt_hbm, rew_hbm, subt_hbm, act_hbm, out_hbm,
             xtbuf, actrep, zrow, rew_v, sub_v, in_sem, out_sem):
    wid = lax.axis_index("c") * 16 + lax.axis_index("s")
    base = wid * ROWS_W

    pltpu.sync_copy(rew_hbm.at[pl.ds(base, ROWS_W)], rew_v)
    pltpu.sync_copy(subt_hbm.at[:, pl.ds(base, ROWS_W)], sub_v)

    def zr_body(v, _):
        zrow[pl.ds(v * 16, 16)] = jnp.zeros((16,), jnp.float32)
        return 0
    lax.fori_loop(0, DU // 16, zr_body, 0)

    pltpu.sync_copy(act_hbm, actrep)

    def stage(c, slot):
        row0 = base + c * CH
        return pltpu.async_copy(
            xt_hbm.at[pl.ds(row0, CH)], xtbuf.at[slot], in_sem)

    def fire(c, slot):
        row0 = base + c * CH
        handles = []
        for h in range(HEADS):
            handles.append(pltpu.async_copy(
                xtbuf.at[slot],
                out_hbm.at[h, pl.ds(row0, CH), pl.ds(0, DU)], out_sem))
            handles.append(pltpu.async_copy(
                actrep.at[h],
                out_hbm.at[h, pl.ds(row0, CH), pl.ds(DU, DA)], out_sem))
        return handles

    pending = [None, None]
    stage(0, 0).wait()
    pending[0] = fire(0, 0)
    for c in range(1, NCH):
        slot = c % 2
        if pending[slot] is not None:
            for hnd in pending[slot]:
                hnd.wait()
            pending[slot] = None
        stage(c, slot).wait()
        pending[slot] = fire(c, slot)
    for p in pending:
        if p is not None:
            for hnd in p:
                hnd.wait()

    def sweep(i, _):
        c = lax.div(i, HEADS)
        h = lax.rem(i, HEADS)
        off = c * CH
        row0 = base + off
        rew_a = rew_v[pl.ds(off, 16)]
        rew_b = rew_v[pl.ds(off + 16, 16)]
        sub_a = sub_v[h, pl.ds(off, 16)]
        sub_b = sub_v[h, pl.ds(off + 16, 16)]
        mfa = jnp.where((rew_a == 1) & (sub_a >= 0.1), 1.0, 0.0)
        mfb = jnp.where((rew_b == 1) & (sub_b >= 0.1), 1.0, 0.0)
        nsel = jnp.sum(mfa) + jnp.sum(mfb)

        @pl.when(nsel < CH - 0.5)
        def _fix():
            def rowfn(r, _):
                lane = lax.rem(r, 16)
                sel_a = jnp.max(
                    jnp.where(lax.iota(jnp.int32, 16) == lane, mfa, 0.0))
                sel_b = jnp.max(
                    jnp.where(lax.iota(jnp.int32, 16) == lane, mfb, 0.0))
                mr = jnp.where(r < 16, sel_a, sel_b)

                @pl.when(mr < 0.5)
                def _zero():
                    pltpu.sync_copy(
                        zrow, out_hbm.at[h, row0 + r, pl.ds(0, DU)])
                return 0
            lax.fori_loop(0, CH, rowfn, 0)
        return 0

    lax.fori_loop(0, NCH * HEADS, sweep, 0)


_sc_call = functools.partial(
    pl.kernel,
    out_type=jax.ShapeDtypeStruct((HEADS, B, DU + DA), jnp.float32),
    mesh=plsc.VectorSubcoreMesh(core_axis_name="c", subcore_axis_name="s"),
    compiler_params=pltpu.CompilerParams(needs_layout_passes=False),
    scratch_types=[
        pltpu.VMEM((2, CH, DU), jnp.float32),
        pltpu.VMEM((HEADS, CH, DA), jnp.float32),
        pltpu.VMEM((DU,), jnp.float32),
        pltpu.VMEM((ROWS_W,), jnp.int32),
        pltpu.VMEM((HEADS, ROWS_W), jnp.float32),
        pltpu.SemaphoreType.DMA,
        pltpu.SemaphoreType.DMA,
    ],
)(_sc_body)


def kernel(xt, rewards, subset, action):
    xt2 = xt.reshape(B, DU)
    subt = subset.T
    actrep = jnp.broadcast_to(action[:, None, :], (HEADS, CH, DA))
    out = _sc_call(xt2, rewards, subt, actrep)
    return out.reshape(HEADS, B, 1, DU + DA)
